# baseline (speedup 1.0000x reference)
import jax
import jax.numpy as jnp
from jax.experimental import pallas as pl

_BATCH = 16
_NC = 81
_NB = 15130
_NBPAD = 15232
_MAXN = 200
_CRIT = 0.45


def _decode_softmax_body(db_ref, bb_ref, sc_ref, boxes_ref, probs_ref):
    s = sc_ref[0]
    m = jnp.max(s, axis=0, keepdims=True)
    e = jnp.exp(s - m)
    denom = jnp.sum(e, axis=0, keepdims=True)
    probs_ref[0] = e[1:] / denom

    bb = bb_ref[0]
    db = db_ref[...]
    xy0 = 0.1 * bb[0:1] * db[2:3] + db[0:1]
    xy1 = 0.1 * bb[1:2] * db[3:4] + db[1:2]
    wh0 = jnp.exp(0.2 * bb[2:3]) * db[2:3]
    wh1 = jnp.exp(0.2 * bb[3:4]) * db[3:4]
    l = xy0 - 0.5 * wh0
    t = xy1 - 0.5 * wh1
    r = xy0 + 0.5 * wh0
    btm = xy1 + 0.5 * wh1
    boxes_ref[0] = jnp.concatenate([l, t, r, btm], axis=0)


def _decode_softmax(bboxes_in, scores_in, dboxes_xywh):
    pad = _NBPAD - _NB
    bb = jnp.pad(bboxes_in, ((0, 0), (0, 0), (0, pad)))
    sc = jnp.pad(scores_in, ((0, 0), (0, 0), (0, pad)), constant_values=-1e30)
    db = jnp.pad(jnp.transpose(dboxes_xywh, (1, 0)), ((0, 0), (0, pad)))
    boxes, probs = pl.pallas_call(
        _decode_softmax_body,
        grid=(_BATCH,),
        in_specs=[
            pl.BlockSpec((4, _NBPAD), lambda b: (0, 0)),
            pl.BlockSpec((1, 4, _NBPAD), lambda b: (b, 0, 0)),
            pl.BlockSpec((1, _NC, _NBPAD), lambda b: (b, 0, 0)),
        ],
        out_specs=[
            pl.BlockSpec((1, 4, _NBPAD), lambda b: (b, 0, 0)),
            pl.BlockSpec((1, _NC - 1, _NBPAD), lambda b: (b, 0, 0)),
        ],
        out_shape=[
            jax.ShapeDtypeStruct((_BATCH, 4, _NBPAD), jnp.float32),
            jax.ShapeDtypeStruct((_BATCH, _NC - 1, _NBPAD), jnp.float32),
        ],
    )(db, bb, sc)
    return boxes[:, :, :_NB], probs[:, :, :_NB]


def _iou_one_many(box, boxes):
    lt = jnp.maximum(box[:2], boxes[:, :2])
    rb = jnp.minimum(box[2:], boxes[:, 2:])
    delta = jnp.clip(rb - lt, 0.0)
    inter = delta[:, 0] * delta[:, 1]
    a1 = (box[2] - box[0]) * (box[3] - box[1])
    d2 = boxes[:, 2:] - boxes[:, :2]
    a2 = d2[:, 0] * d2[:, 1]
    return inter / (a1 + a2 - inter)


def _nms_one(boxes, scores):
    s = jnp.where(scores > 0.05, scores, -jnp.inf)
    top_s, top_i = jax.lax.top_k(s, _MAXN)
    b = boxes[top_i]
    keep0 = top_s > 0.05
    idxs = jnp.arange(_MAXN)

    def body(i, keep):
        iou = _iou_one_many(b[i], b)
        suppress = (iou >= _CRIT) & (idxs > i) & keep[i]
        return keep & (~suppress)

    keep = jax.lax.fori_loop(0, _MAXN, body, keep0)
    out_s = jnp.where(keep, top_s, -jnp.inf)
    return b, out_s


def kernel(bboxes_in, scores_in, dboxes_xywh):
    boxes4, cls_probs = _decode_softmax(bboxes_in, scores_in, dboxes_xywh)
    boxes = jnp.transpose(boxes4, (0, 2, 1))
    nms_c = jax.vmap(_nms_one, in_axes=(None, 0))
    nms_b = jax.vmap(nms_c, in_axes=(0, 0))
    b_out, s_out = nms_b(boxes, cls_probs)
    B = _BATCH
    C1 = _NC - 1
    labels = jnp.broadcast_to(
        (jnp.arange(C1, dtype=jnp.int32) + 1)[None, :, None], (B, C1, _MAXN))
    s_flat = s_out.reshape(B, C1 * _MAXN)
    b_flat = b_out.reshape(B, C1 * _MAXN, 4)
    l_flat = labels.reshape(B, C1 * _MAXN)
    top_s, top_i = jax.lax.top_k(s_flat, _MAXN)
    boxes_out = jnp.take_along_axis(b_flat, top_i[..., None], axis=1)
    labels_out = jnp.take_along_axis(l_flat, top_i, axis=1)
    scores_out = jnp.where(jnp.isfinite(top_s), top_s, 0.0)
    return boxes_out, scores_out, labels_out


# speedup vs baseline: 1.1546x; 1.1546x over previous
import jax
import jax.numpy as jnp
from jax.experimental import pallas as pl

_BATCH = 16
_NC = 81
_NB = 15130
_NBPAD = 15232
_MAXN = 200
_CRIT = 0.45
_TH = 0.05
_CB = 16


def _decode_softmax_body(db_ref, bb_ref, sc_ref, boxes_ref, probs_ref):
    s = sc_ref[0]
    m = jnp.max(s, axis=0, keepdims=True)
    e = jnp.exp(s - m)
    denom = jnp.sum(e, axis=0, keepdims=True)
    p = e[1:] / denom
    probs_ref[0] = jnp.where(p > _TH, p, -jnp.inf)

    bb = bb_ref[0]
    db = db_ref[...]
    xy0 = 0.1 * bb[0:1] * db[2:3] + db[0:1]
    xy1 = 0.1 * bb[1:2] * db[3:4] + db[1:2]
    wh0 = jnp.exp(0.2 * bb[2:3]) * db[2:3]
    wh1 = jnp.exp(0.2 * bb[3:4]) * db[3:4]
    l = xy0 - 0.5 * wh0
    t = xy1 - 0.5 * wh1
    r = xy0 + 0.5 * wh0
    btm = xy1 + 0.5 * wh1
    boxes_ref[0] = jnp.concatenate([l, t, r, btm], axis=0)


def _decode_softmax(bboxes_in, scores_in, dboxes_xywh):
    pad = _NBPAD - _NB
    bb = jnp.pad(bboxes_in, ((0, 0), (0, 0), (0, pad)))
    sc = jnp.pad(scores_in, ((0, 0), (0, 0), (0, pad)), constant_values=-1e30)
    db = jnp.pad(jnp.transpose(dboxes_xywh, (1, 0)), ((0, 0), (0, pad)))
    boxes, probs = pl.pallas_call(
        _decode_softmax_body,
        grid=(_BATCH,),
        in_specs=[
            pl.BlockSpec((4, _NBPAD), lambda b: (0, 0)),
            pl.BlockSpec((1, 4, _NBPAD), lambda b: (b, 0, 0)),
            pl.BlockSpec((1, _NC, _NBPAD), lambda b: (b, 0, 0)),
        ],
        out_specs=[
            pl.BlockSpec((1, 4, _NBPAD), lambda b: (b, 0, 0)),
            pl.BlockSpec((1, _NC - 1, _NBPAD), lambda b: (b, 0, 0)),
        ],
        out_shape=[
            jax.ShapeDtypeStruct((_BATCH, 4, _NBPAD), jnp.float32),
            jax.ShapeDtypeStruct((_BATCH, _NC - 1, _NBPAD), jnp.float32),
        ],
    )(db, bb, sc)
    return boxes, probs


def _nms_body(l_ref, t_ref, r_ref, b_ref, s_ref, out_ref):
    l = l_ref[0]
    t = t_ref[0]
    r = r_ref[0]
    b = b_ref[0]
    s = s_ref[0]
    area = (r - l) * (b - t)

    li = jnp.transpose(l)[:, :, None]
    ti = jnp.transpose(t)[:, :, None]
    ri = jnp.transpose(r)[:, :, None]
    bi = jnp.transpose(b)[:, :, None]
    ai = jnp.transpose(area)[:, :, None]
    lj = l[None]
    tj = t[None]
    rj = r[None]
    bj = b[None]
    aj = area[None]
    dx = jnp.maximum(jnp.minimum(ri, rj) - jnp.maximum(li, lj), 0.0)
    dy = jnp.maximum(jnp.minimum(bi, bj) - jnp.maximum(ti, tj), 0.0)
    inter = dx * dy
    iou = inter / (ai + aj - inter)
    ii = jax.lax.broadcasted_iota(jnp.int32, (_MAXN, 1, _MAXN), 0)
    jj = jax.lax.broadcasted_iota(jnp.int32, (_MAXN, 1, _MAXN), 2)
    M = jnp.where((iou >= _CRIT) & (jj > ii), 1.0, 0.0)

    keep = jnp.where(s > _TH, 1.0, 0.0)
    for i in range(_MAXN):
        ki = keep[:, i:i + 1]
        keep = keep * (1.0 - M[i] * ki)
    out_ref[0] = jnp.where(keep > 0.0, s, -jnp.inf)


def _nms(bl, bt, br, bb2, ts):
    spec = pl.BlockSpec((1, _CB, _MAXN), lambda b, c: (b, c, 0))
    return pl.pallas_call(
        _nms_body,
        grid=(_BATCH, (_NC - 1) // _CB),
        in_specs=[spec] * 5,
        out_specs=spec,
        out_shape=jax.ShapeDtypeStruct((_BATCH, _NC - 1, _MAXN), jnp.float32),
    )(bl, bt, br, bb2, ts)


def kernel(bboxes_in, scores_in, dboxes_xywh):
    boxes4, probs = _decode_softmax(bboxes_in, scores_in, dboxes_xywh)

    top_s, top_i = jax.lax.top_k(probs, _MAXN)
    bl = jnp.take_along_axis(boxes4[:, 0:1, :], top_i, axis=2)
    bt = jnp.take_along_axis(boxes4[:, 1:2, :], top_i, axis=2)
    br = jnp.take_along_axis(boxes4[:, 2:3, :], top_i, axis=2)
    bb2 = jnp.take_along_axis(boxes4[:, 3:4, :], top_i, axis=2)

    out_s = _nms(bl, bt, br, bb2, top_s)

    s_flat = out_s.reshape(_BATCH, -1)
    top_s2, top_i2 = jax.lax.top_k(s_flat, _MAXN)
    gl = jnp.take_along_axis(bl.reshape(_BATCH, -1), top_i2, axis=1)
    gt = jnp.take_along_axis(bt.reshape(_BATCH, -1), top_i2, axis=1)
    gr = jnp.take_along_axis(br.reshape(_BATCH, -1), top_i2, axis=1)
    gb = jnp.take_along_axis(bb2.reshape(_BATCH, -1), top_i2, axis=1)
    boxes_out = jnp.stack([gl, gt, gr, gb], axis=-1)
    labels_out = (top_i2 // _MAXN + 1).astype(jnp.int32)
    scores_out = jnp.where(jnp.isfinite(top_s2), top_s2, 0.0)
    return boxes_out, scores_out, labels_out


# speedup vs baseline: 1.3826x; 1.1974x over previous
import jax
import jax.numpy as jnp
from jax.experimental import pallas as pl

_BATCH = 16
_NC = 81
_NB = 15130
_NBPAD = 15232
_MAXN = 200
_CRIT = 0.45
_TH = 0.05
_CB = 16

_NROWS = _BATCH * (_NC - 1)
_NW = 32
_RPW = _NROWS // _NW
_CAP = 1024
_FILL = 208
_CAND = _CAP + _FILL
_BUF = _CAND + 16
_NCHUNK = _NBPAD // 16
_NSPLIT = 17


def _decode_softmax_body(db_ref, bb_ref, sc_ref, boxes_ref, probs_ref):
    s = sc_ref[0]
    m = jnp.max(s, axis=0, keepdims=True)
    e = jnp.exp(s - m)
    denom = jnp.sum(e, axis=0, keepdims=True)
    p = e[1:] / denom
    probs_ref[0] = jnp.where(p > _TH, p, -jnp.inf)

    bb = bb_ref[0]
    db = db_ref[...]
    xy0 = 0.1 * bb[0:1] * db[2:3] + db[0:1]
    xy1 = 0.1 * bb[1:2] * db[3:4] + db[1:2]
    wh0 = jnp.exp(0.2 * bb[2:3]) * db[2:3]
    wh1 = jnp.exp(0.2 * bb[3:4]) * db[3:4]
    l = xy0 - 0.5 * wh0
    t = xy1 - 0.5 * wh1
    r = xy0 + 0.5 * wh0
    btm = xy1 + 0.5 * wh1
    boxes_ref[0] = jnp.concatenate([l, t, r, btm], axis=0)


def _decode_softmax(bboxes_in, scores_in, dboxes_xywh):
    pad = _NBPAD - _NB
    bb = jnp.pad(bboxes_in, ((0, 0), (0, 0), (0, pad)))
    sc = jnp.pad(scores_in, ((0, 0), (0, 0), (0, pad)), constant_values=-1e30)
    db = jnp.pad(jnp.transpose(dboxes_xywh, (1, 0)), ((0, 0), (0, pad)))
    boxes, probs = pl.pallas_call(
        _decode_softmax_body,
        grid=(_BATCH,),
        in_specs=[
            pl.BlockSpec((4, _NBPAD), lambda b: (0, 0)),
            pl.BlockSpec((1, 4, _NBPAD), lambda b: (b, 0, 0)),
            pl.BlockSpec((1, _NC, _NBPAD), lambda b: (b, 0, 0)),
        ],
        out_specs=[
            pl.BlockSpec((1, 4, _NBPAD), lambda b: (b, 0, 0)),
            pl.BlockSpec((1, _NC - 1, _NBPAD), lambda b: (b, 0, 0)),
        ],
        out_shape=[
            jax.ShapeDtypeStruct((_BATCH, 4, _NBPAD), jnp.float32),
            jax.ShapeDtypeStruct((_BATCH, _NC - 1, _NBPAD), jnp.float32),
        ],
    )(db, bb, sc)
    return boxes, probs


def _nms_body(l_ref, t_ref, r_ref, b_ref, s_ref, out_ref):
    l = l_ref[0]
    t = t_ref[0]
    r = r_ref[0]
    b = b_ref[0]
    s = s_ref[0]
    area = (r - l) * (b - t)

    li = jnp.transpose(l)[:, :, None]
    ti = jnp.transpose(t)[:, :, None]
    ri = jnp.transpose(r)[:, :, None]
    bi = jnp.transpose(b)[:, :, None]
    ai = jnp.transpose(area)[:, :, None]
    lj = l[None]
    tj = t[None]
    rj = r[None]
    bj = b[None]
    aj = area[None]
    dx = jnp.maximum(jnp.minimum(ri, rj) - jnp.maximum(li, lj), 0.0)
    dy = jnp.maximum(jnp.minimum(bi, bj) - jnp.maximum(ti, tj), 0.0)
    inter = dx * dy
    iou = inter / (ai + aj - inter)
    ii = jax.lax.broadcasted_iota(jnp.int32, (_MAXN, 1, _MAXN), 0)
    jj = jax.lax.broadcasted_iota(jnp.int32, (_MAXN, 1, _MAXN), 2)
    M = jnp.where((iou >= _CRIT) & (jj > ii), 1.0, 0.0)

    keep = jnp.where(s > _TH, 1.0, 0.0)
    for i in range(_MAXN):
        ki = keep[:, i:i + 1]
        keep = keep * (1.0 - M[i] * ki)
    out_ref[0] = jnp.where(keep > 0.0, s, -jnp.inf)


def _nms(bl, bt, br, bb2, ts):
    spec = pl.BlockSpec((1, _CB, _MAXN), lambda b, c: (b, c, 0))
    return pl.pallas_call(
        _nms_body,
        grid=(_BATCH, (_NC - 1) // _CB),
        in_specs=[spec] * 5,
        out_specs=spec,
        out_shape=jax.ShapeDtypeStruct((_BATCH, _NC - 1, _MAXN), jnp.float32),
    )(bl, bt, br, bb2, ts)


def kernel(bboxes_in, scores_in, dboxes_xywh):
    boxes4, probs = _decode_softmax(bboxes_in, scores_in, dboxes_xywh)

    pc = probs.reshape(_NROWS, _NSPLIT, _NBPAD // _NSPLIT)
    s1, i1 = jax.lax.top_k(pc, _MAXN)
    base = (jnp.arange(_NSPLIT, dtype=jnp.int32) * (_NBPAD // _NSPLIT))
    g1 = (i1 + base[None, :, None]).reshape(_NROWS, _NSPLIT * _MAXN)
    top_s, top_p = jax.lax.top_k(s1.reshape(_NROWS, -1), _MAXN)
    top_i = jnp.take_along_axis(g1, top_p, axis=1)
    top_s = top_s.reshape(_BATCH, _NC - 1, _MAXN)
    top_i = top_i.reshape(_BATCH, _NC - 1, _MAXN)
    bl = jnp.take_along_axis(boxes4[:, 0:1, :], top_i, axis=2)
    bt = jnp.take_along_axis(boxes4[:, 1:2, :], top_i, axis=2)
    br = jnp.take_along_axis(boxes4[:, 2:3, :], top_i, axis=2)
    bb2 = jnp.take_along_axis(boxes4[:, 3:4, :], top_i, axis=2)

    out_s = _nms(bl, bt, br, bb2, top_s)

    s_flat = out_s.reshape(_BATCH, -1)
    top_s2, top_i2 = jax.lax.top_k(s_flat, _MAXN)
    gl = jnp.take_along_axis(bl.reshape(_BATCH, -1), top_i2, axis=1)
    gt = jnp.take_along_axis(bt.reshape(_BATCH, -1), top_i2, axis=1)
    gr = jnp.take_along_axis(br.reshape(_BATCH, -1), top_i2, axis=1)
    gb = jnp.take_along_axis(bb2.reshape(_BATCH, -1), top_i2, axis=1)
    boxes_out = jnp.stack([gl, gt, gr, gb], axis=-1)
    labels_out = (top_i2 // _MAXN + 1).astype(jnp.int32)
    scores_out = jnp.where(jnp.isfinite(top_s2), top_s2, 0.0)
    return boxes_out, scores_out, labels_out
